# Initial kernel scaffold; baseline (speedup 1.0000x reference)
import jax
import jax.numpy as jnp
from jax.experimental import pallas as pl


def kernel(x_Patient, x_Admission, edges, params):
    raise NotImplementedError("write your pallas kernel here")


# speedup vs baseline: 4.1456x; 4.1456x over previous
import functools
import math

import jax
import jax.numpy as jnp
from jax import lax
from jax.experimental import pallas as pl
from jax.experimental.pallas import tpu as pltpu
from jax.experimental.pallas import tpu_sc as plsc

HIDDEN = 64
NC = 2
NS = 16
LANES = 16
CHUNK = 128
NBUF = 4
EGROUP = NS * CHUNK * NBUF

_NTYPE_ORDER = ["Patient", "Admission", "Diagnosis", "Medication", "Procedure", "LabTest"]
_REL_LIST = [
    ("Patient", "Admission"),
    ("Admission", "Patient"),
    ("Admission", "Diagnosis"),
    ("Diagnosis", "Admission"),
    ("Admission", "Medication"),
    ("Medication", "Admission"),
    ("Admission", "Procedure"),
    ("Procedure", "Admission"),
    ("Admission", "LabTest"),
    ("LabTest", "Admission"),
]


def _rkey(s, d):
    return s + "__" + d


def _round_up(x, m):
    return ((x + m - 1) // m) * m


def _npass(n_dst):
    return 1 if n_dst // NC <= 20000 else 2


def _acc_height(part):
    return _round_up(part + NS * LANES, NS * 8)


def _fill_buffer(buf, value):
    val = jnp.full((LANES,), value, jnp.float32)

    @pl.loop(0, CHUNK)
    def _(i):
        for j in range(HIDDEN // LANES):
            buf[i, pl.ds(j * LANES, LANES)] = val


def _zero_acc(acc, zbuf, sid, acc_h):
    zpt = acc_h // NS
    nfull = zpt // CHUNK
    tail = zpt - nfull * CHUNK

    @pl.loop(0, nfull)
    def _(k):
        pltpu.sync_copy(zbuf, acc.at[pl.ds(sid * zpt + k * CHUNK, CHUNK)])

    if tail:
        pltpu.sync_copy(zbuf.at[pl.ds(0, tail)],
                        acc.at[pl.ds(sid * zpt + nfull * CHUNK, tail)])


def _copy_out(acc, out_hbm, sid, h_lo, half):
    nfull = half // CHUNK
    tail = half - nfull * CHUNK

    @pl.loop(sid, nfull, step=NS)
    def _(j):
        pltpu.sync_copy(acc.at[pl.ds(j * CHUNK, CHUNK)],
                        out_hbm.at[pl.ds(h_lo + j * CHUNK, CHUNK)])

    if tail:
        @pl.when(sid == NS - 1)
        def _():
            pltpu.sync_copy(acc.at[pl.ds(nfull * CHUNK, tail)],
                            out_hbm.at[pl.ds(h_lo + nfull * CHUNK, tail)])


@functools.cache
def _segsum_kernel(n_src, n_dst, e_pad):
    del n_src
    npass = _npass(n_dst)
    part = n_dst // (NC * npass)
    acc_h = _acc_height(part)
    cpt = e_pad // (NS * CHUNK)
    assert cpt % NBUF == 0 and n_dst % (NC * npass) == 0

    mesh = plsc.VectorSubcoreMesh(core_axis_name="c", subcore_axis_name="s",
                                  num_cores=NC, num_subcores=NS)
    scratch = (
        [pltpu.VMEM((CHUNK,), jnp.int32) for _ in range(NBUF)]
        + [pltpu.VMEM((CHUNK,), jnp.int32) for _ in range(NBUF)]
        + [pltpu.VMEM((CHUNK, HIDDEN), jnp.float32) for _ in range(NBUF)]
        + [pltpu.VMEM((CHUNK, HIDDEN), jnp.float32)]
        + [pltpu.VMEM_SHARED((acc_h, HIDDEN), jnp.float32)]
        + [pltpu.SemaphoreType.DMA for _ in range(3 * NBUF)]
    )

    def body(src_hbm, dst_hbm, x_hbm, out_hbm, *rest):
        srcidx = rest[0:NBUF]
        dstidx = rest[NBUF:2 * NBUF]
        rows = rest[2 * NBUF:3 * NBUF]
        zbuf = rest[3 * NBUF]
        acc = rest[3 * NBUF + 1]
        sem_i = rest[3 * NBUF + 2: 3 * NBUF + 2 + NBUF]
        sem_g = rest[3 * NBUF + 2 + NBUF: 3 * NBUF + 2 + 2 * NBUF]
        sem_s = rest[3 * NBUF + 2 + 2 * NBUF:]

        cid = lax.axis_index("c")
        sid = lax.axis_index("s")
        dummy = (acc_h - NS * LANES) + sid * LANES + lax.iota(jnp.int32, LANES)

        _fill_buffer(zbuf, 0.0)
        for pss in range(npass):
            h_lo = (pss * NC + cid) * part
            _zero_acc(acc, zbuf, sid, acc_h)
            plsc.subcore_barrier()

            @pl.loop(0, cpt, step=NBUF)
            def _(j0):
                for b in range(NBUF):
                    base = (sid * cpt + j0 + b) * CHUNK
                    pltpu.make_async_copy(src_hbm.at[pl.ds(base, CHUNK)], srcidx[b], sem_i[b]).start()
                    pltpu.make_async_copy(dst_hbm.at[pl.ds(base, CHUNK)], dstidx[b], sem_i[b]).start()
                for b in range(NBUF):
                    base = (sid * cpt + j0 + b) * CHUNK
                    pltpu.make_async_copy(src_hbm.at[pl.ds(base, CHUNK)], srcidx[b], sem_i[b]).wait()
                    pltpu.make_async_copy(dst_hbm.at[pl.ds(base, CHUNK)], dstidx[b], sem_i[b]).wait()
                    for k in range(CHUNK // LANES):
                        d = dstidx[b][pl.ds(k * LANES, LANES)]
                        loc = d - h_lo
                        ok = (loc >= 0) & (loc < part)
                        dstidx[b][pl.ds(k * LANES, LANES)] = jnp.where(ok, loc, dummy)
                    pltpu.make_async_copy(x_hbm.at[srcidx[b]], rows[b], sem_g[b]).start()
                for b in range(NBUF):
                    pltpu.make_async_copy(x_hbm.at[srcidx[b]], rows[b], sem_g[b]).wait()
                    pltpu.make_async_copy(rows[b], acc.at[dstidx[b]], sem_s[b]).start(add=True)
                for b in range(NBUF):
                    pltpu.make_async_copy(rows[b], acc.at[dstidx[b]], sem_s[b]).wait()

            plsc.subcore_barrier()
            _copy_out(acc, out_hbm, sid, h_lo, part)
            if npass > 1:
                plsc.subcore_barrier()

    return pl.kernel(
        body,
        out_type=jax.ShapeDtypeStruct((n_dst, HIDDEN), jnp.float32),
        mesh=mesh,
        scratch_types=scratch,
        compiler_params=pltpu.CompilerParams(use_tc_tiling_on_sc=False),
        name=f"segsum_{n_dst}_{e_pad}",
    )


@functools.cache
def _count_kernel(n_dst, e_pad):
    npass = _npass(n_dst)
    part = n_dst // (NC * npass)
    acc_h = _acc_height(part)
    cpt = e_pad // (NS * CHUNK)
    assert cpt % NBUF == 0 and n_dst % (NC * npass) == 0

    mesh = plsc.VectorSubcoreMesh(core_axis_name="c", subcore_axis_name="s",
                                  num_cores=NC, num_subcores=NS)
    scratch = (
        [pltpu.VMEM((CHUNK,), jnp.int32) for _ in range(NBUF)]
        + [pltpu.VMEM((CHUNK, HIDDEN), jnp.float32)]
        + [pltpu.VMEM((CHUNK, HIDDEN), jnp.float32)]
        + [pltpu.VMEM_SHARED((acc_h, HIDDEN), jnp.float32)]
        + [pltpu.SemaphoreType.DMA for _ in range(2 * NBUF)]
    )

    def body(dst_hbm, out_hbm, *rest):
        dstidx = rest[0:NBUF]
        ones = rest[NBUF]
        zbuf = rest[NBUF + 1]
        acc = rest[NBUF + 2]
        sem_i = rest[NBUF + 3: NBUF + 3 + NBUF]
        sem_s = rest[NBUF + 3 + NBUF:]

        cid = lax.axis_index("c")
        sid = lax.axis_index("s")
        dummy = (acc_h - NS * LANES) + sid * LANES + lax.iota(jnp.int32, LANES)

        _fill_buffer(zbuf, 0.0)
        _fill_buffer(ones, 1.0)
        for pss in range(npass):
            h_lo = (pss * NC + cid) * part
            _zero_acc(acc, zbuf, sid, acc_h)
            plsc.subcore_barrier()

            @pl.loop(0, cpt, step=NBUF)
            def _(j0):
                for b in range(NBUF):
                    base = (sid * cpt + j0 + b) * CHUNK
                    pltpu.make_async_copy(dst_hbm.at[pl.ds(base, CHUNK)], dstidx[b], sem_i[b]).start()
                for b in range(NBUF):
                    base = (sid * cpt + j0 + b) * CHUNK
                    pltpu.make_async_copy(dst_hbm.at[pl.ds(base, CHUNK)], dstidx[b], sem_i[b]).wait()
                    for k in range(CHUNK // LANES):
                        d = dstidx[b][pl.ds(k * LANES, LANES)]
                        loc = d - h_lo
                        ok = (loc >= 0) & (loc < part)
                        dstidx[b][pl.ds(k * LANES, LANES)] = jnp.where(ok, loc, dummy)
                    pltpu.make_async_copy(ones, acc.at[dstidx[b]], sem_s[b]).start(add=True)
                for b in range(NBUF):
                    pltpu.make_async_copy(ones, acc.at[dstidx[b]], sem_s[b]).wait()

            plsc.subcore_barrier()
            _copy_out(acc, out_hbm, sid, h_lo, part)
            if npass > 1:
                plsc.subcore_barrier()

    return pl.kernel(
        body,
        out_type=jax.ShapeDtypeStruct((n_dst, HIDDEN), jnp.float32),
        mesh=mesh,
        scratch_types=scratch,
        compiler_params=pltpu.CompilerParams(use_tc_tiling_on_sc=False),
        name=f"count_{n_dst}_{e_pad}",
    )


def _segsum(srcp, dstp, x_src, n_dst):
    return _segsum_kernel(x_src.shape[0], n_dst, srcp.shape[0])(srcp, dstp, x_src)


def _count(dstp, n_dst):
    return _count_kernel(n_dst, dstp.shape[0])(dstp)


_BLK = 512
_BN_SCALE = 1.0 / math.sqrt(1.0 + 1e-5)


@functools.cache
def _encoder_kernel(n, d_in_pad):
    grid = (pl.cdiv(n, _BLK),)

    def body(x_ref, w_ref, b_ref, g_ref, bb_ref, out_ref):
        y = jnp.dot(x_ref[...], w_ref[...], preferred_element_type=jnp.float32)
        y = y + b_ref[...]
        out_ref[...] = y * (g_ref[...] * _BN_SCALE) + bb_ref[...]

    return pl.pallas_call(
        body,
        grid=grid,
        in_specs=[
            pl.BlockSpec((_BLK, d_in_pad), lambda i: (i, 0)),
            pl.BlockSpec((d_in_pad, HIDDEN), lambda i: (0, 0)),
            pl.BlockSpec((1, HIDDEN), lambda i: (0, 0)),
            pl.BlockSpec((1, HIDDEN), lambda i: (0, 0)),
            pl.BlockSpec((1, HIDDEN), lambda i: (0, 0)),
        ],
        out_specs=pl.BlockSpec((_BLK, HIDDEN), lambda i: (i, 0)),
        out_shape=jax.ShapeDtypeStruct((n, HIDDEN), jnp.float32),
        name=f"encode_{n}_{d_in_pad}",
    )


def _encode(x, w, b, g, bb):
    n, d_in = x.shape
    d_pad = _round_up(d_in, 128)
    xp = jnp.pad(x, ((0, 0), (0, d_pad - d_in)))
    wp = jnp.pad(w, ((0, d_pad - d_in), (0, 0)))
    return _encoder_kernel(n, d_pad)(xp, wp, b.reshape(1, -1), g.reshape(1, -1),
                                     bb.reshape(1, -1))


@functools.cache
def _combine_kernel(n_dst, nrel, mode):
    grid = (pl.cdiv(n_dst, _BLK),)
    blk = pl.BlockSpec((_BLK, HIDDEN), lambda i: (i, 0))
    mat = pl.BlockSpec((HIDDEN, HIDDEN), lambda i: (0, 0))
    vec = pl.BlockSpec((1, HIDDEN), lambda i: (0, 0))

    in_specs = [blk]
    for _ in range(nrel):
        in_specs += [blk, blk, mat, mat, vec]
    if mode == "bn":
        in_specs += [vec, vec]
        out_w = HIDDEN
    else:
        in_specs += [pl.BlockSpec((HIDDEN, 128), lambda i: (0, 0)),
                     pl.BlockSpec((1, 128), lambda i: (0, 0))]
        out_w = 128

    def body(*refs):
        x_ref = refs[0]
        out_ref = refs[-1]
        x = x_ref[...]
        acc = None
        for r in range(nrel):
            seg, cnt, wl, wr, bl = refs[1 + 5 * r: 6 + 5 * r]
            aggr = seg[...] / jnp.maximum(cnt[...], 1.0)
            o = (jnp.dot(aggr, wl[...], preferred_element_type=jnp.float32)
                 + bl[...]
                 + jnp.dot(x, wr[...], preferred_element_type=jnp.float32))
            acc = o if acc is None else jnp.maximum(acc, o)
        if mode == "bn":
            g_ref, b_ref = refs[1 + 5 * nrel: 3 + 5 * nrel]
            out_ref[...] = jax.nn.relu(acc * (g_ref[...] * _BN_SCALE) + b_ref[...])
        else:
            w_ref, b_ref = refs[1 + 5 * nrel: 3 + 5 * nrel]
            h = jax.nn.relu(acc)
            out_ref[...] = (jnp.dot(h, w_ref[...], preferred_element_type=jnp.float32)
                            + b_ref[...])

    return pl.pallas_call(
        body,
        grid=grid,
        in_specs=in_specs,
        out_specs=pl.BlockSpec((_BLK, out_w), lambda i: (i, 0)),
        out_shape=jax.ShapeDtypeStruct((n_dst, out_w), jnp.float32),
        name=f"combine_{n_dst}_{nrel}_{mode}",
    )


def kernel(x_Patient, x_Admission, edges, params):
    p = params
    sizes = {"Patient": x_Patient.shape[0], "Admission": x_Admission.shape[0]}
    for nt in ("Diagnosis", "Medication", "Procedure", "LabTest"):
        sizes[nt] = p["emb"][nt].shape[0]

    x = {
        "Patient": _encode(x_Patient, p["pat_lin"]["W"], p["pat_lin"]["b"],
                           p["pat_bn"]["g"], p["pat_bn"]["b"]),
        "Admission": _encode(x_Admission, p["adm_lin"]["W"], p["adm_lin"]["b"],
                             p["adm_bn"]["g"], p["adm_bn"]["b"]),
        "Diagnosis": p["emb"]["Diagnosis"],
        "Medication": p["emb"]["Medication"],
        "Procedure": p["emb"]["Procedure"],
        "LabTest": p["emb"]["LabTest"],
    }

    srcp, dstp = {}, {}
    for (s, d) in _REL_LIST:
        k = _rkey(s, d)
        e = edges[k].shape[1]
        e_pad = _round_up(e, EGROUP)
        pad = e_pad - e
        fill_src = (jnp.arange(pad, dtype=jnp.int32) % sizes[s])
        srcp[k] = jnp.concatenate([edges[k][0], fill_src])
        dstp[k] = jnp.concatenate(
            [edges[k][1], jnp.full((pad,), sizes[d], jnp.int32)])

    cnt = {}
    for (s, d) in _REL_LIST:
        k = _rkey(s, d)
        cnt[k] = _count(dstp[k], sizes[d])

    rels_by_dst = {}
    for (s, d) in _REL_LIST:
        rels_by_dst.setdefault(d, []).append((s, d))

    out = {}
    for li, l in enumerate(("1", "2", "3")):
        seg = {}
        for (s, d) in _REL_LIST:
            k = _rkey(s, d)
            seg[k] = _segsum(srcp[k], dstp[k], x[s], sizes[d])
        newx = {}
        for d, rels in rels_by_dst.items():
            n_d = sizes[d]
            args = [x[d]]
            for (s, _) in rels:
                k = _rkey(s, d)
                pc = p["conv"][l][k]
                args += [seg[k], cnt[k], pc["Wl"], pc["Wr"], pc["bl"].reshape(1, -1)]
            if li < 2:
                bn = p["bn"][l][d]
                args += [bn["g"].reshape(1, -1), bn["b"].reshape(1, -1)]
                newx[d] = _combine_kernel(n_d, len(rels), "bn")(*args)
            else:
                wh = jnp.pad(p["lin"][d]["W"], ((0, 0), (0, 128 - p["lin"][d]["W"].shape[1])))
                bh = jnp.pad(p["lin"][d]["b"], (0, 128 - p["lin"][d]["b"].shape[0]))
                args += [wh, bh.reshape(1, -1)]
                out[d] = _combine_kernel(n_d, len(rels), "head")(*args)
        x = newx

    nout = p["lin"]["Patient"]["W"].shape[1]
    return tuple(out[nt][:, :nout] for nt in _NTYPE_ORDER)


# speedup vs baseline: 4.4086x; 1.0634x over previous
import functools
import math

import jax
import jax.numpy as jnp
from jax import lax
from jax.experimental import pallas as pl
from jax.experimental.pallas import tpu as pltpu
from jax.experimental.pallas import tpu_sc as plsc

HIDDEN = 64
NC = 2
NS = 16
LANES = 16
CHUNK = 128
NBUF = 4
RAW = 1024
EGROUP = NS * RAW

_NTYPE_ORDER = ["Patient", "Admission", "Diagnosis", "Medication", "Procedure", "LabTest"]
_REL_LIST = [
    ("Patient", "Admission"),
    ("Admission", "Patient"),
    ("Admission", "Diagnosis"),
    ("Diagnosis", "Admission"),
    ("Admission", "Medication"),
    ("Medication", "Admission"),
    ("Admission", "Procedure"),
    ("Procedure", "Admission"),
    ("Admission", "LabTest"),
    ("LabTest", "Admission"),
]


def _rkey(s, d):
    return s + "__" + d


def _round_up(x, m):
    return ((x + m - 1) // m) * m


def _npass(n_dst):
    return 1 if n_dst // NC <= 20000 else 2


def _acc_height(part):
    return _round_up(part + NS * LANES, NS * 8)


def _fill_buffer(buf, value):
    val = jnp.full((LANES,), value, jnp.float32)

    @pl.loop(0, CHUNK)
    def _(i):
        for j in range(HIDDEN // LANES):
            buf[i, pl.ds(j * LANES, LANES)] = val


def _zero_acc(acc, zbuf, sid, acc_h):
    zpt = acc_h // NS
    nfull = zpt // CHUNK
    tail = zpt - nfull * CHUNK

    @pl.loop(0, nfull)
    def _(k):
        pltpu.sync_copy(zbuf, acc.at[pl.ds(sid * zpt + k * CHUNK, CHUNK)])

    if tail:
        pltpu.sync_copy(zbuf.at[pl.ds(0, tail)],
                        acc.at[pl.ds(sid * zpt + nfull * CHUNK, tail)])


def _copy_out(acc, out_hbm, sid, h_lo, half):
    nfull = half // CHUNK
    tail = half - nfull * CHUNK

    @pl.loop(sid, nfull, step=NS)
    def _(j):
        pltpu.sync_copy(acc.at[pl.ds(j * CHUNK, CHUNK)],
                        out_hbm.at[pl.ds(h_lo + j * CHUNK, CHUNK)])

    if tail:
        @pl.when(sid == NS - 1)
        def _():
            pltpu.sync_copy(acc.at[pl.ds(nfull * CHUNK, tail)],
                            out_hbm.at[pl.ds(h_lo + nfull * CHUNK, tail)])


@functools.cache
def _segsum_kernel(n_src, n_dst, e_pad):
    del n_src
    npass = _npass(n_dst)
    part = n_dst // (NC * npass)
    acc_h = _acc_height(part)
    nraw = e_pad // (NS * RAW)
    assert e_pad % (NS * RAW) == 0 and n_dst % (NC * npass) == 0

    mesh = plsc.VectorSubcoreMesh(core_axis_name="c", subcore_axis_name="s",
                                  num_cores=NC, num_subcores=NS)
    scratch = (
        [pltpu.VMEM((2, RAW), jnp.int32)]
        + [pltpu.VMEM((2, RAW), jnp.int32)]
        + [pltpu.VMEM((RAW + LANES,), jnp.int32)]
        + [pltpu.VMEM((RAW + LANES,), jnp.int32)]
        + [pltpu.VMEM((CHUNK,), jnp.int32) for _ in range(NBUF)]
        + [pltpu.VMEM((CHUNK,), jnp.int32) for _ in range(NBUF)]
        + [pltpu.VMEM((CHUNK, HIDDEN), jnp.float32) for _ in range(NBUF)]
        + [pltpu.VMEM((CHUNK, HIDDEN), jnp.float32)]
        + [pltpu.VMEM_SHARED((acc_h, HIDDEN), jnp.float32)]
        + [pltpu.SemaphoreType.DMA for _ in range(2 + 2 * NBUF)]
    )

    def body(src_hbm, dst_hbm, x_hbm, out_hbm, *rest):
        sraw, draw, cs, cd = rest[0:4]
        fsrc = rest[4:4 + NBUF]
        fdst = rest[4 + NBUF:4 + 2 * NBUF]
        rows = rest[4 + 2 * NBUF:4 + 3 * NBUF]
        zbuf = rest[4 + 3 * NBUF]
        acc = rest[5 + 3 * NBUF]
        sem_r = rest[6 + 3 * NBUF:8 + 3 * NBUF]
        sem_g = rest[8 + 3 * NBUF:8 + 3 * NBUF + NBUF]
        sem_s = rest[8 + 3 * NBUF + NBUF:]

        cid = lax.axis_index("c")
        sid = lax.axis_index("s")
        iota = lax.iota(jnp.int32, LANES)
        dummy = (acc_h - NS * LANES) + sid * LANES + iota
        tile_base = sid * (e_pad // NS)

        def raw_slice(rc):
            return pl.ds(tile_base + rc * RAW, RAW)

        def wait_prev_scatter(bb):
            pltpu.make_async_copy(rows[bb], acc.at[fdst[bb]], sem_s[bb]).wait()

        def finish_fire(bb):
            pltpu.make_async_copy(x_hbm.at[fsrc[bb]], rows[bb], sem_g[bb]).wait()
            pltpu.make_async_copy(rows[bb], acc.at[fdst[bb]], sem_s[bb]).start(add=True)

        _fill_buffer(zbuf, 0.0)
        for pss in range(npass):
            h_lo = (pss * NC + cid) * part
            _zero_acc(acc, zbuf, sid, acc_h)
            plsc.subcore_barrier()

            pltpu.make_async_copy(src_hbm.at[raw_slice(0)], sraw.at[0], sem_r[0]).start()
            pltpu.make_async_copy(dst_hbm.at[raw_slice(0)], draw.at[0], sem_r[0]).start()

            def raw_body(rc, nf):
                for rbs in range(2):
                    @pl.when((rc & 1) == rbs)
                    def _():
                        @pl.when(rc + 1 < nraw)
                        def _():
                            pltpu.make_async_copy(src_hbm.at[raw_slice(rc + 1)],
                                                  sraw.at[1 - rbs], sem_r[1 - rbs]).start()
                            pltpu.make_async_copy(dst_hbm.at[raw_slice(rc + 1)],
                                                  draw.at[1 - rbs], sem_r[1 - rbs]).start()
                        pltpu.make_async_copy(src_hbm.at[raw_slice(rc)],
                                              sraw.at[rbs], sem_r[rbs]).wait()
                        pltpu.make_async_copy(dst_hbm.at[raw_slice(rc)],
                                              draw.at[rbs], sem_r[rbs]).wait()
                rb = rc & 1

                def scan_g(g, m):
                    off = g * LANES
                    sv = sraw[rb, pl.ds(off, LANES)]
                    dv = draw[rb, pl.ds(off, LANES)]
                    loc = dv - h_lo
                    ok = (loc >= 0) & (loc < part)
                    pos = m + plsc.cumsum(ok.astype(jnp.int32)) - 1
                    plsc.store_scatter(cs, [pos], sv, mask=ok)
                    plsc.store_scatter(cd, [pos], loc, mask=ok)
                    return m + jnp.sum(ok.astype(jnp.int32))

                mc = pl.loop(0, RAW // LANES, init_carry=jnp.int32(0))(scan_g)
                mpad = ((mc + CHUNK - 1) >> 7) << 7

                def pad_g(g, _):
                    off = g * LANES
                    pos = off + iota
                    keep = pos < mc
                    cs[pl.ds(off, LANES)] = jnp.where(keep, cs[pl.ds(off, LANES)], iota)
                    cd[pl.ds(off, LANES)] = jnp.where(keep, cd[pl.ds(off, LANES)], dummy)
                    return 0

                pl.loop(mc >> 4, mpad >> 4, init_carry=jnp.int32(0))(pad_g)

                def fire_q(q, nf):
                    qoff = q * CHUNK
                    for bb in range(NBUF):
                        @pl.when((nf & (NBUF - 1)) == bb)
                        def _():
                            @pl.when(nf >= NBUF)
                            def _():
                                wait_prev_scatter(bb)
                            for g in range(CHUNK // LANES):
                                fsrc[bb][pl.ds(g * LANES, LANES)] = cs[pl.ds(qoff + g * LANES, LANES)]
                                fdst[bb][pl.ds(g * LANES, LANES)] = cd[pl.ds(qoff + g * LANES, LANES)]
                            pltpu.make_async_copy(x_hbm.at[fsrc[bb]], rows[bb], sem_g[bb]).start()
                    pv = nf - 1

                    @pl.when(pv >= 0)
                    def _():
                        for bb in range(NBUF):
                            @pl.when((pv & (NBUF - 1)) == bb)
                            def _():
                                finish_fire(bb)
                    return nf + 1

                return pl.loop(0, mpad >> 7, init_carry=nf)(fire_q)

            nf = pl.loop(0, nraw, init_carry=jnp.int32(0))(raw_body)

            @pl.when(nf >= 1)
            def _():
                pv = nf - 1
                for bb in range(NBUF):
                    @pl.when((pv & (NBUF - 1)) == bb)
                    def _():
                        finish_fire(bb)

            for bb in range(NBUF):
                @pl.when(bb < nf)
                def _():
                    wait_prev_scatter(bb)

            plsc.subcore_barrier()
            _copy_out(acc, out_hbm, sid, h_lo, part)
            if npass > 1:
                plsc.subcore_barrier()

    return pl.kernel(
        body,
        out_type=jax.ShapeDtypeStruct((n_dst, HIDDEN), jnp.float32),
        mesh=mesh,
        scratch_types=scratch,
        compiler_params=pltpu.CompilerParams(use_tc_tiling_on_sc=False, needs_layout_passes=False),
        name=f"segsum_{n_dst}_{e_pad}",
    )


@functools.cache
def _count_kernel(n_dst, e_pad):
    npass = _npass(n_dst)
    part = n_dst // (NC * npass)
    acc_h = _acc_height(part)
    nraw = e_pad // (NS * RAW)
    assert e_pad % (NS * RAW) == 0 and n_dst % (NC * npass) == 0

    mesh = plsc.VectorSubcoreMesh(core_axis_name="c", subcore_axis_name="s",
                                  num_cores=NC, num_subcores=NS)
    scratch = (
        [pltpu.VMEM((2, RAW), jnp.int32)]
        + [pltpu.VMEM((RAW + LANES,), jnp.int32)]
        + [pltpu.VMEM((CHUNK,), jnp.int32) for _ in range(NBUF)]
        + [pltpu.VMEM((CHUNK, HIDDEN), jnp.float32)]
        + [pltpu.VMEM((CHUNK, HIDDEN), jnp.float32)]
        + [pltpu.VMEM_SHARED((acc_h, HIDDEN), jnp.float32)]
        + [pltpu.SemaphoreType.DMA for _ in range(2 + NBUF)]
    )

    def body(dst_hbm, out_hbm, *rest):
        draw = rest[0]
        cd = rest[1]
        fdst = rest[2:2 + NBUF]
        ones = rest[2 + NBUF]
        zbuf = rest[3 + NBUF]
        acc = rest[4 + NBUF]
        sem_r = rest[5 + NBUF:7 + NBUF]
        sem_s = rest[7 + NBUF:]

        cid = lax.axis_index("c")
        sid = lax.axis_index("s")
        iota = lax.iota(jnp.int32, LANES)
        dummy = (acc_h - NS * LANES) + sid * LANES + iota
        tile_base = sid * (e_pad // NS)

        def raw_slice(rc):
            return pl.ds(tile_base + rc * RAW, RAW)

        def wait_scatter(bb):
            pltpu.make_async_copy(ones, acc.at[fdst[bb]], sem_s[bb]).wait()

        _fill_buffer(zbuf, 0.0)
        _fill_buffer(ones, 1.0)
        for pss in range(npass):
            h_lo = (pss * NC + cid) * part
            _zero_acc(acc, zbuf, sid, acc_h)
            plsc.subcore_barrier()

            pltpu.make_async_copy(dst_hbm.at[raw_slice(0)], draw.at[0], sem_r[0]).start()

            def raw_body(rc, nf):
                for rbs in range(2):
                    @pl.when((rc & 1) == rbs)
                    def _():
                        @pl.when(rc + 1 < nraw)
                        def _():
                            pltpu.make_async_copy(dst_hbm.at[raw_slice(rc + 1)],
                                                  draw.at[1 - rbs], sem_r[1 - rbs]).start()
                        pltpu.make_async_copy(dst_hbm.at[raw_slice(rc)],
                                              draw.at[rbs], sem_r[rbs]).wait()
                rb = rc & 1

                def scan_g(g, m):
                    dv = draw[rb, pl.ds(g * LANES, LANES)]
                    loc = dv - h_lo
                    ok = (loc >= 0) & (loc < part)
                    pos = m + plsc.cumsum(ok.astype(jnp.int32)) - 1
                    plsc.store_scatter(cd, [pos], loc, mask=ok)
                    return m + jnp.sum(ok.astype(jnp.int32))

                mc = pl.loop(0, RAW // LANES, init_carry=jnp.int32(0))(scan_g)
                mpad = ((mc + CHUNK - 1) >> 7) << 7

                def pad_g(g, _):
                    off = g * LANES
                    keep = (off + iota) < mc
                    cd[pl.ds(off, LANES)] = jnp.where(keep, cd[pl.ds(off, LANES)], dummy)
                    return 0

                pl.loop(mc >> 4, mpad >> 4, init_carry=jnp.int32(0))(pad_g)

                def fire_q(q, nf):
                    qoff = q * CHUNK
                    for bb in range(NBUF):
                        @pl.when((nf & (NBUF - 1)) == bb)
                        def _():
                            @pl.when(nf >= NBUF)
                            def _():
                                wait_scatter(bb)
                            for g in range(CHUNK // LANES):
                                fdst[bb][pl.ds(g * LANES, LANES)] = cd[pl.ds(qoff + g * LANES, LANES)]
                            pltpu.make_async_copy(ones, acc.at[fdst[bb]], sem_s[bb]).start(add=True)
                    return nf + 1

                return pl.loop(0, mpad >> 7, init_carry=nf)(fire_q)

            nf = pl.loop(0, nraw, init_carry=jnp.int32(0))(raw_body)

            for bb in range(NBUF):
                @pl.when(bb < nf)
                def _():
                    wait_scatter(bb)

            plsc.subcore_barrier()
            _copy_out(acc, out_hbm, sid, h_lo, part)
            if npass > 1:
                plsc.subcore_barrier()

    return pl.kernel(
        body,
        out_type=jax.ShapeDtypeStruct((n_dst, HIDDEN), jnp.float32),
        mesh=mesh,
        scratch_types=scratch,
        compiler_params=pltpu.CompilerParams(use_tc_tiling_on_sc=False, needs_layout_passes=False),
        name=f"count_{n_dst}_{e_pad}",
    )


def _segsum(srcp, dstp, x_src, n_dst):
    return _segsum_kernel(x_src.shape[0], n_dst, srcp.shape[0])(srcp, dstp, x_src)


def _count(dstp, n_dst):
    return _count_kernel(n_dst, dstp.shape[0])(dstp)


_BLK = 512
_BN_SCALE = 1.0 / math.sqrt(1.0 + 1e-5)


@functools.cache
def _encoder_kernel(n, d_in_pad):
    grid = (pl.cdiv(n, _BLK),)

    def body(x_ref, w_ref, b_ref, g_ref, bb_ref, out_ref):
        y = jnp.dot(x_ref[...], w_ref[...], preferred_element_type=jnp.float32)
        y = y + b_ref[...]
        out_ref[...] = y * (g_ref[...] * _BN_SCALE) + bb_ref[...]

    return pl.pallas_call(
        body,
        grid=grid,
        in_specs=[
            pl.BlockSpec((_BLK, d_in_pad), lambda i: (i, 0)),
            pl.BlockSpec((d_in_pad, HIDDEN), lambda i: (0, 0)),
            pl.BlockSpec((1, HIDDEN), lambda i: (0, 0)),
            pl.BlockSpec((1, HIDDEN), lambda i: (0, 0)),
            pl.BlockSpec((1, HIDDEN), lambda i: (0, 0)),
        ],
        out_specs=pl.BlockSpec((_BLK, HIDDEN), lambda i: (i, 0)),
        out_shape=jax.ShapeDtypeStruct((n, HIDDEN), jnp.float32),
        name=f"encode_{n}_{d_in_pad}",
    )


def _encode(x, w, b, g, bb):
    n, d_in = x.shape
    d_pad = _round_up(d_in, 128)
    xp = jnp.pad(x, ((0, 0), (0, d_pad - d_in)))
    wp = jnp.pad(w, ((0, d_pad - d_in), (0, 0)))
    return _encoder_kernel(n, d_pad)(xp, wp, b.reshape(1, -1), g.reshape(1, -1),
                                     bb.reshape(1, -1))


@functools.cache
def _combine_kernel(n_dst, nrel, mode):
    grid = (pl.cdiv(n_dst, _BLK),)
    blk = pl.BlockSpec((_BLK, HIDDEN), lambda i: (i, 0))
    mat = pl.BlockSpec((HIDDEN, HIDDEN), lambda i: (0, 0))
    vec = pl.BlockSpec((1, HIDDEN), lambda i: (0, 0))

    in_specs = [blk]
    for _ in range(nrel):
        in_specs += [blk, blk, mat, mat, vec]
    if mode == "bn":
        in_specs += [vec, vec]
        out_w = HIDDEN
    else:
        in_specs += [pl.BlockSpec((HIDDEN, 128), lambda i: (0, 0)),
                     pl.BlockSpec((1, 128), lambda i: (0, 0))]
        out_w = 128

    def body(*refs):
        x_ref = refs[0]
        out_ref = refs[-1]
        x = x_ref[...]
        acc = None
        for r in range(nrel):
            seg, cnt, wl, wr, bl = refs[1 + 5 * r: 6 + 5 * r]
            aggr = seg[...] / jnp.maximum(cnt[...], 1.0)
            o = (jnp.dot(aggr, wl[...], preferred_element_type=jnp.float32)
                 + bl[...]
                 + jnp.dot(x, wr[...], preferred_element_type=jnp.float32))
            acc = o if acc is None else jnp.maximum(acc, o)
        if mode == "bn":
            g_ref, b_ref = refs[1 + 5 * nrel: 3 + 5 * nrel]
            out_ref[...] = jax.nn.relu(acc * (g_ref[...] * _BN_SCALE) + b_ref[...])
        else:
            w_ref, b_ref = refs[1 + 5 * nrel: 3 + 5 * nrel]
            h = jax.nn.relu(acc)
            out_ref[...] = (jnp.dot(h, w_ref[...], preferred_element_type=jnp.float32)
                            + b_ref[...])

    return pl.pallas_call(
        body,
        grid=grid,
        in_specs=in_specs,
        out_specs=pl.BlockSpec((_BLK, out_w), lambda i: (i, 0)),
        out_shape=jax.ShapeDtypeStruct((n_dst, out_w), jnp.float32),
        name=f"combine_{n_dst}_{nrel}_{mode}",
    )


def kernel(x_Patient, x_Admission, edges, params):
    p = params
    sizes = {"Patient": x_Patient.shape[0], "Admission": x_Admission.shape[0]}
    for nt in ("Diagnosis", "Medication", "Procedure", "LabTest"):
        sizes[nt] = p["emb"][nt].shape[0]

    x = {
        "Patient": _encode(x_Patient, p["pat_lin"]["W"], p["pat_lin"]["b"],
                           p["pat_bn"]["g"], p["pat_bn"]["b"]),
        "Admission": _encode(x_Admission, p["adm_lin"]["W"], p["adm_lin"]["b"],
                             p["adm_bn"]["g"], p["adm_bn"]["b"]),
        "Diagnosis": p["emb"]["Diagnosis"],
        "Medication": p["emb"]["Medication"],
        "Procedure": p["emb"]["Procedure"],
        "LabTest": p["emb"]["LabTest"],
    }

    srcp, dstp = {}, {}
    for (s, d) in _REL_LIST:
        k = _rkey(s, d)
        e = edges[k].shape[1]
        e_pad = _round_up(e, EGROUP)
        pad = e_pad - e
        fill_src = (jnp.arange(pad, dtype=jnp.int32) % sizes[s])
        srcp[k] = jnp.concatenate([edges[k][0], fill_src])
        dstp[k] = jnp.concatenate(
            [edges[k][1], jnp.full((pad,), sizes[d], jnp.int32)])

    cnt = {}
    for (s, d) in _REL_LIST:
        k = _rkey(s, d)
        cnt[k] = _count(dstp[k], sizes[d])

    rels_by_dst = {}
    for (s, d) in _REL_LIST:
        rels_by_dst.setdefault(d, []).append((s, d))

    out = {}
    for li, l in enumerate(("1", "2", "3")):
        seg = {}
        for (s, d) in _REL_LIST:
            k = _rkey(s, d)
            seg[k] = _segsum(srcp[k], dstp[k], x[s], sizes[d])
        newx = {}
        for d, rels in rels_by_dst.items():
            n_d = sizes[d]
            args = [x[d]]
            for (s, _) in rels:
                k = _rkey(s, d)
                pc = p["conv"][l][k]
                args += [seg[k], cnt[k], pc["Wl"], pc["Wr"], pc["bl"].reshape(1, -1)]
            if li < 2:
                bn = p["bn"][l][d]
                args += [bn["g"].reshape(1, -1), bn["b"].reshape(1, -1)]
                newx[d] = _combine_kernel(n_d, len(rels), "bn")(*args)
            else:
                wh = jnp.pad(p["lin"][d]["W"], ((0, 0), (0, 128 - p["lin"][d]["W"].shape[1])))
                bh = jnp.pad(p["lin"][d]["b"], (0, 128 - p["lin"][d]["b"].shape[0]))
                args += [wh, bh.reshape(1, -1)]
                out[d] = _combine_kernel(n_d, len(rels), "head")(*args)
        x = newx

    nout = p["lin"]["Patient"]["W"].shape[1]
    return tuple(out[nt][:, :nout] for nt in _NTYPE_ORDER)


# speedup vs baseline: 5.8047x; 1.3167x over previous
import functools
import math

import jax
import jax.numpy as jnp
from jax import lax
from jax.experimental import pallas as pl
from jax.experimental.pallas import tpu as pltpu
from jax.experimental.pallas import tpu_sc as plsc

HIDDEN = 64
NC = 2
NS = 16
LANES = 16
CHUNK = 128
NBUF = 4
RAW = 1024
EGROUP = NS * RAW * 2

_SC_PARAMS = pltpu.CompilerParams(use_tc_tiling_on_sc=False,
                                  needs_layout_passes=False)

_NTYPE_ORDER = ["Patient", "Admission", "Diagnosis", "Medication", "Procedure", "LabTest"]
_REL_LIST = [
    ("Patient", "Admission"),
    ("Admission", "Patient"),
    ("Admission", "Diagnosis"),
    ("Diagnosis", "Admission"),
    ("Admission", "Medication"),
    ("Medication", "Admission"),
    ("Admission", "Procedure"),
    ("Procedure", "Admission"),
    ("Admission", "LabTest"),
    ("LabTest", "Admission"),
]


def _rkey(s, d):
    return s + "__" + d


def _round_up(x, m):
    return ((x + m - 1) // m) * m


def _geom(n_dst, e_pad):
    npass = 1 if n_dst // NC <= 20000 else 2
    part = n_dst // (NC * npass)
    acc_h = _round_up(part + NS * LANES, NS * 8)
    cap = e_pad // NS
    slots = npass * NC * NS
    assert n_dst % (NC * npass) == 0 and cap % CHUNK == 0
    return npass, part, acc_h, cap, slots


def _mesh():
    return plsc.VectorSubcoreMesh(core_axis_name="c", subcore_axis_name="s",
                                  num_cores=NC, num_subcores=NS)


def _fill_buffer(buf, value):
    val = jnp.full((LANES,), value, jnp.float32)

    @pl.loop(0, CHUNK)
    def _(i):
        for j in range(HIDDEN // LANES):
            buf[i, pl.ds(j * LANES, LANES)] = val


def _zero_acc(acc, zbuf, sid, acc_h):
    zpt = acc_h // NS
    nfull = zpt // CHUNK
    tail = zpt - nfull * CHUNK

    @pl.loop(0, nfull)
    def _(k):
        pltpu.sync_copy(zbuf, acc.at[pl.ds(sid * zpt + k * CHUNK, CHUNK)])

    if tail:
        pltpu.sync_copy(zbuf.at[pl.ds(0, tail)],
                        acc.at[pl.ds(sid * zpt + nfull * CHUNK, tail)])


def _copy_out(acc, out_hbm, sid, h_lo, part):
    nfull = part // CHUNK
    tail = part - nfull * CHUNK

    @pl.loop(sid, nfull, step=NS)
    def _(j):
        pltpu.sync_copy(acc.at[pl.ds(j * CHUNK, CHUNK)],
                        out_hbm.at[pl.ds(h_lo + j * CHUNK, CHUNK)])

    if tail:
        @pl.when(sid == NS - 1)
        def _():
            pltpu.sync_copy(acc.at[pl.ds(nfull * CHUNK, tail)],
                            out_hbm.at[pl.ds(h_lo + nfull * CHUNK, tail)])


@functools.cache
def _partition_kernel(n_dst, e_pad):
    npass, part, acc_h, cap, slots = _geom(n_dst, e_pad)
    nraw = e_pad // (NS * RAW)
    assert nraw % 2 == 0
    STG = RAW + LANES

    out_type = (
        jax.ShapeDtypeStruct((slots * cap,), jnp.int32),
        jax.ShapeDtypeStruct((slots * cap,), jnp.int32),
        jax.ShapeDtypeStruct((slots, LANES), jnp.int32),
    )
    scratch = (
        [pltpu.VMEM((2, RAW), jnp.int32)]
        + [pltpu.VMEM((2, RAW), jnp.int32)]
        + [pltpu.VMEM((2, STG), jnp.int32)]
        + [pltpu.VMEM((2, STG), jnp.int32)]
        + [pltpu.VMEM((LANES,), jnp.int32)]
        + [pltpu.SemaphoreType.DMA for _ in range(4)]
    )

    def body(src_hbm, dst_hbm, tok, psrc, pdst, pcnt, *rest):
        del tok
        sraw, draw, cs2, cd2, cntv = rest[0:5]
        sem_i = rest[5:7]
        sem_w = rest[7:9]

        cid = lax.axis_index("c")
        sid = lax.axis_index("s")
        iota = lax.iota(jnp.int32, LANES)
        dummy_s = sid * LANES + iota
        tile_base = sid * (e_pad // NS)

        def raw_slice(rc):
            return pl.ds(tile_base + rc * RAW, RAW)

        def start_idx(rc, rbs):
            pltpu.make_async_copy(src_hbm.at[raw_slice(rc)], sraw.at[rbs], sem_i[rbs]).start()
            pltpu.make_async_copy(dst_hbm.at[raw_slice(rc)], draw.at[rbs], sem_i[rbs]).start()

        def wait_idx(rc, rbs):
            pltpu.make_async_copy(src_hbm.at[raw_slice(rc)], sraw.at[rbs], sem_i[rbs]).wait()
            pltpu.make_async_copy(dst_hbm.at[raw_slice(rc)], draw.at[rbs], sem_i[rbs]).wait()

        def wait_fire_pair(rbs):
            pltpu.make_async_copy(cs2.at[rbs, pl.ds(0, CHUNK)],
                                  psrc.at[pl.ds(0, CHUNK)], sem_w[rbs]).wait()
            pltpu.make_async_copy(cd2.at[rbs, pl.ds(0, CHUNK)],
                                  pdst.at[pl.ds(0, CHUNK)], sem_w[rbs]).wait()

        for pss in range(npass):
            h_lo = (pss * NC + cid) * part
            slot = (pss * NC + cid) * NS + sid
            sbase = slot * cap
            dummy_d = (acc_h - NS * LANES) + sid * LANES + iota

            start_idx(0, 0)
            start_idx(1, 1)

            def process(rc, rbs, nb, npv):
                wait_idx(rc, rbs)

                @pl.loop(0, npv)
                def _(i):
                    wait_fire_pair(rbs)

                def scan_g(g, m):
                    off = g * LANES
                    sv = sraw[rbs, pl.ds(off, LANES)]
                    dv = draw[rbs, pl.ds(off, LANES)]
                    loc = dv - h_lo
                    ok = (loc >= 0) & (loc < part)
                    c = plsc.cumsum(ok.astype(jnp.int32))
                    pos = m + c - 1
                    plsc.store_scatter(cs2.at[rbs], [pos], sv, mask=ok)
                    plsc.store_scatter(cd2.at[rbs], [pos], loc, mask=ok)
                    return m + c[LANES - 1]

                mc = pl.loop(0, RAW // LANES, init_carry=jnp.int32(0))(scan_g)
                mpad = ((mc + CHUNK - 1) >> 7) << 7

                def pad_g(g, _):
                    off = g * LANES
                    keep = (off + iota) < mc
                    cs2[rbs, pl.ds(off, LANES)] = jnp.where(
                        keep, cs2[rbs, pl.ds(off, LANES)], dummy_s)
                    cd2[rbs, pl.ds(off, LANES)] = jnp.where(
                        keep, cd2[rbs, pl.ds(off, LANES)], dummy_d)
                    return 0

                pl.loop(mc >> 4, mpad >> 4, init_carry=jnp.int32(0))(pad_g)
                nq = mpad >> 7

                @pl.loop(0, nq)
                def _(q):
                    pltpu.make_async_copy(
                        cs2.at[rbs, pl.ds(q * CHUNK, CHUNK)],
                        psrc.at[pl.ds(sbase + (nb + q) * CHUNK, CHUNK)],
                        sem_w[rbs]).start()
                    pltpu.make_async_copy(
                        cd2.at[rbs, pl.ds(q * CHUNK, CHUNK)],
                        pdst.at[pl.ds(sbase + (nb + q) * CHUNK, CHUNK)],
                        sem_w[rbs]).start()

                @pl.when(rc + 2 < nraw)
                def _():
                    start_idx(rc + 2, rbs)

                return nq

            def pair_body(rc0, carry):
                nb, np0, np1 = carry
                nq0 = process(rc0, 0, nb, np0)
                nq1 = process(rc0 + 1, 1, nb + nq0, np1)
                return (nb + nq0 + nq1, nq0, nq1)

            nb, np0, np1 = pl.loop(
                0, nraw, step=2,
                init_carry=(jnp.int32(0), jnp.int32(0), jnp.int32(0)))(pair_body)

            @pl.loop(0, np0)
            def _(i):
                wait_fire_pair(0)

            @pl.loop(0, np1)
            def _(i):
                wait_fire_pair(1)

            cntv[pl.ds(0, LANES)] = jnp.zeros((LANES,), jnp.int32) + nb
            pltpu.make_async_copy(cntv, pcnt.at[slot], sem_w[0]).start()
            pltpu.make_async_copy(cntv, pcnt.at[slot], sem_w[0]).wait()

    return pl.kernel(
        body,
        out_type=out_type,
        mesh=_mesh(),
        scratch_types=scratch,
        compiler_params=_SC_PARAMS,
        name=f"edgepart_{n_dst}_{e_pad}",
    )


@functools.cache
def _segsum_kernel(n_src, n_dst, e_pad):
    del n_src
    npass, part, acc_h, cap, slots = _geom(n_dst, e_pad)

    scratch = (
        [pltpu.VMEM((CHUNK,), jnp.int32) for _ in range(NBUF)]
        + [pltpu.VMEM((CHUNK,), jnp.int32) for _ in range(NBUF)]
        + [pltpu.VMEM((CHUNK, HIDDEN), jnp.float32) for _ in range(NBUF)]
        + [pltpu.VMEM((CHUNK, HIDDEN), jnp.float32)]
        + [pltpu.VMEM((LANES,), jnp.int32)]
        + [pltpu.VMEM_SHARED((acc_h, HIDDEN), jnp.float32)]
        + [pltpu.SemaphoreType.DMA for _ in range(3 * NBUF)]
    )

    def body(psrc, pdst, pcnt, x_hbm, tok, out_hbm, *rest):
        del tok
        fsrc = rest[0:NBUF]
        fdst = rest[NBUF:2 * NBUF]
        rows = rest[2 * NBUF:3 * NBUF]
        zbuf = rest[3 * NBUF]
        cntv = rest[3 * NBUF + 1]
        acc = rest[3 * NBUF + 2]
        sem_i = rest[3 * NBUF + 3:3 * NBUF + 3 + NBUF]
        sem_g = rest[3 * NBUF + 3 + NBUF:3 * NBUF + 3 + 2 * NBUF]
        sem_s = rest[3 * NBUF + 3 + 2 * NBUF:]

        cid = lax.axis_index("c")
        sid = lax.axis_index("s")

        _fill_buffer(zbuf, 0.0)
        for pss in range(npass):
            h_lo = (pss * NC + cid) * part
            slot = (pss * NC + cid) * NS + sid
            sbase = slot * cap
            _zero_acc(acc, zbuf, sid, acc_h)
            plsc.subcore_barrier()

            pltpu.sync_copy(pcnt.at[slot], cntv)
            nb = cntv[pl.ds(0, LANES)][0]

            def on_slot(qq, fn):
                for bb in range(NBUF):
                    @pl.when((qq & (NBUF - 1)) == bb)
                    def _():
                        fn(bb)

            def start_idx(qq):
                def f(bb):
                    pltpu.make_async_copy(psrc.at[pl.ds(sbase + qq * CHUNK, CHUNK)],
                                          fsrc[bb], sem_i[bb]).start()
                    pltpu.make_async_copy(pdst.at[pl.ds(sbase + qq * CHUNK, CHUNK)],
                                          fdst[bb], sem_i[bb]).start()
                on_slot(qq, f)

            def start_gather(qq):
                def f(bb):
                    pltpu.make_async_copy(psrc.at[pl.ds(sbase + qq * CHUNK, CHUNK)],
                                          fsrc[bb], sem_i[bb]).wait()
                    pltpu.make_async_copy(pdst.at[pl.ds(sbase + qq * CHUNK, CHUNK)],
                                          fdst[bb], sem_i[bb]).wait()
                    pltpu.make_async_copy(x_hbm.at[fsrc[bb]], rows[bb], sem_g[bb]).start()
                on_slot(qq, f)

            def start_scatter(qq):
                def f(bb):
                    pltpu.make_async_copy(x_hbm.at[fsrc[bb]], rows[bb], sem_g[bb]).wait()
                    pltpu.make_async_copy(rows[bb], acc.at[fdst[bb]], sem_s[bb]).start(add=True)
                on_slot(qq, f)

            def wait_scatter(bb):
                pltpu.make_async_copy(rows[bb], acc.at[fdst[bb]], sem_s[bb]).wait()

            @pl.loop(0, nb)
            def _(q):
                @pl.when(q >= NBUF)
                def _():
                    on_slot(q, wait_scatter)
                start_idx(q)

                @pl.when(q >= 1)
                def _():
                    start_gather(q - 1)

                @pl.when(q >= 2)
                def _():
                    start_scatter(q - 2)

            @pl.when(nb >= 1)
            def _():
                start_gather(nb - 1)

            @pl.when(nb >= 2)
            def _():
                start_scatter(nb - 2)

            @pl.when(nb >= 1)
            def _():
                start_scatter(nb - 1)

            for bb in range(NBUF):
                @pl.when(bb < nb)
                def _():
                    wait_scatter(bb)

            plsc.subcore_barrier()
            _copy_out(acc, out_hbm, sid, h_lo, part)
            if npass > 1:
                plsc.subcore_barrier()

    return pl.kernel(
        body,
        out_type=jax.ShapeDtypeStruct((n_dst, HIDDEN), jnp.float32),
        mesh=_mesh(),
        scratch_types=scratch,
        compiler_params=_SC_PARAMS,
        name=f"segsum_{n_dst}_{e_pad}",
    )


@functools.cache
def _count_kernel(n_dst, e_pad):
    npass, part, acc_h, cap, slots = _geom(n_dst, e_pad)

    scratch = (
        [pltpu.VMEM((CHUNK,), jnp.int32) for _ in range(NBUF)]
        + [pltpu.VMEM((CHUNK, HIDDEN), jnp.float32)]
        + [pltpu.VMEM((CHUNK, HIDDEN), jnp.float32)]
        + [pltpu.VMEM((LANES,), jnp.int32)]
        + [pltpu.VMEM_SHARED((acc_h, HIDDEN), jnp.float32)]
        + [pltpu.SemaphoreType.DMA for _ in range(2 * NBUF)]
    )

    def body(pdst, pcnt, tok, out_hbm, *rest):
        del tok
        fdst = rest[0:NBUF]
        ones = rest[NBUF]
        zbuf = rest[NBUF + 1]
        cntv = rest[NBUF + 2]
        acc = rest[NBUF + 3]
        sem_i = rest[NBUF + 4:NBUF + 4 + NBUF]
        sem_s = rest[NBUF + 4 + NBUF:]

        cid = lax.axis_index("c")
        sid = lax.axis_index("s")

        _fill_buffer(zbuf, 0.0)
        _fill_buffer(ones, 1.0)
        for pss in range(npass):
            h_lo = (pss * NC + cid) * part
            slot = (pss * NC + cid) * NS + sid
            sbase = slot * cap
            _zero_acc(acc, zbuf, sid, acc_h)
            plsc.subcore_barrier()

            pltpu.sync_copy(pcnt.at[slot], cntv)
            nb = cntv[pl.ds(0, LANES)][0]

            def on_slot(qq, fn):
                for bb in range(NBUF):
                    @pl.when((qq & (NBUF - 1)) == bb)
                    def _():
                        fn(bb)

            def start_idx(qq):
                def f(bb):
                    pltpu.make_async_copy(pdst.at[pl.ds(sbase + qq * CHUNK, CHUNK)],
                                          fdst[bb], sem_i[bb]).start()
                on_slot(qq, f)

            def start_scatter(qq):
                def f(bb):
                    pltpu.make_async_copy(pdst.at[pl.ds(sbase + qq * CHUNK, CHUNK)],
                                          fdst[bb], sem_i[bb]).wait()
                    pltpu.make_async_copy(ones, acc.at[fdst[bb]], sem_s[bb]).start(add=True)
                on_slot(qq, f)

            def wait_scatter(bb):
                pltpu.make_async_copy(ones, acc.at[fdst[bb]], sem_s[bb]).wait()

            @pl.loop(0, nb)
            def _(q):
                @pl.when(q >= NBUF)
                def _():
                    on_slot(q, wait_scatter)
                start_idx(q)

                @pl.when(q >= 1)
                def _():
                    start_scatter(q - 1)

            @pl.when(nb >= 1)
            def _():
                start_scatter(nb - 1)

            for bb in range(NBUF):
                @pl.when(bb < nb)
                def _():
                    wait_scatter(bb)

            plsc.subcore_barrier()
            _copy_out(acc, out_hbm, sid, h_lo, part)
            if npass > 1:
                plsc.subcore_barrier()

    return pl.kernel(
        body,
        out_type=jax.ShapeDtypeStruct((n_dst, HIDDEN), jnp.float32),
        mesh=_mesh(),
        scratch_types=scratch,
        compiler_params=_SC_PARAMS,
        name=f"count_{n_dst}_{e_pad}",
    )


def _partition(srcp, dstp, n_dst, tok):
    out = _partition_kernel(n_dst, srcp.shape[0])(srcp, dstp, tok)
    return out, out[2][0]


def _segsum(parts, x_src, n_dst, e_pad, tok):
    psrc, pdst, pcnt = parts
    out = _segsum_kernel(x_src.shape[0], n_dst, e_pad)(psrc, pdst, pcnt, x_src, tok)
    return out, out[0, :16].astype(jnp.int32)


def _count(parts, n_dst, e_pad, tok):
    _, pdst, pcnt = parts
    out = _count_kernel(n_dst, e_pad)(pdst, pcnt, tok)
    return out, out[0, :16].astype(jnp.int32)


_BLK = 512
_BN_SCALE = 1.0 / math.sqrt(1.0 + 1e-5)


@functools.cache
def _encoder_kernel(n, d_in_pad):
    grid = (pl.cdiv(n, _BLK),)

    def body(x_ref, w_ref, b_ref, g_ref, bb_ref, out_ref):
        y = jnp.dot(x_ref[...], w_ref[...], preferred_element_type=jnp.float32)
        y = y + b_ref[...]
        out_ref[...] = y * (g_ref[...] * _BN_SCALE) + bb_ref[...]

    return pl.pallas_call(
        body,
        grid=grid,
        in_specs=[
            pl.BlockSpec((_BLK, d_in_pad), lambda i: (i, 0)),
            pl.BlockSpec((d_in_pad, HIDDEN), lambda i: (0, 0)),
            pl.BlockSpec((1, HIDDEN), lambda i: (0, 0)),
            pl.BlockSpec((1, HIDDEN), lambda i: (0, 0)),
            pl.BlockSpec((1, HIDDEN), lambda i: (0, 0)),
        ],
        out_specs=pl.BlockSpec((_BLK, HIDDEN), lambda i: (i, 0)),
        out_shape=jax.ShapeDtypeStruct((n, HIDDEN), jnp.float32),
        name=f"encode_{n}_{d_in_pad}",
    )


def _encode(x, w, b, g, bb):
    n, d_in = x.shape
    d_pad = _round_up(d_in, 128)
    xp = jnp.pad(x, ((0, 0), (0, d_pad - d_in)))
    wp = jnp.pad(w, ((0, d_pad - d_in), (0, 0)))
    return _encoder_kernel(n, d_pad)(xp, wp, b.reshape(1, -1), g.reshape(1, -1),
                                     bb.reshape(1, -1))


@functools.cache
def _combine_kernel(n_dst, nrel, mode):
    grid = (pl.cdiv(n_dst, _BLK),)
    blk = pl.BlockSpec((_BLK, HIDDEN), lambda i: (i, 0))
    mat = pl.BlockSpec((HIDDEN, HIDDEN), lambda i: (0, 0))
    vec = pl.BlockSpec((1, HIDDEN), lambda i: (0, 0))

    in_specs = [blk]
    for _ in range(nrel):
        in_specs += [blk, blk, mat, mat, vec]
    if mode == "bn":
        in_specs += [vec, vec]
        out_w = HIDDEN
    else:
        in_specs += [pl.BlockSpec((HIDDEN, 128), lambda i: (0, 0)),
                     pl.BlockSpec((1, 128), lambda i: (0, 0))]
        out_w = 128

    def body(*refs):
        x_ref = refs[0]
        out_ref = refs[-1]
        x = x_ref[...]
        acc = None
        for r in range(nrel):
            seg, cnt, wl, wr, bl = refs[1 + 5 * r: 6 + 5 * r]
            aggr = seg[...] / jnp.maximum(cnt[...], 1.0)
            o = (jnp.dot(aggr, wl[...], preferred_element_type=jnp.float32)
                 + bl[...]
                 + jnp.dot(x, wr[...], preferred_element_type=jnp.float32))
            acc = o if acc is None else jnp.maximum(acc, o)
        if mode == "bn":
            g_ref, b_ref = refs[1 + 5 * nrel: 3 + 5 * nrel]
            out_ref[...] = jax.nn.relu(acc * (g_ref[...] * _BN_SCALE) + b_ref[...])
        else:
            w_ref, b_ref = refs[1 + 5 * nrel: 3 + 5 * nrel]
            h = jax.nn.relu(acc)
            out_ref[...] = (jnp.dot(h, w_ref[...], preferred_element_type=jnp.float32)
                            + b_ref[...])

    return pl.pallas_call(
        body,
        grid=grid,
        in_specs=in_specs,
        out_specs=pl.BlockSpec((_BLK, out_w), lambda i: (i, 0)),
        out_shape=jax.ShapeDtypeStruct((n_dst, out_w), jnp.float32),
        name=f"combine_{n_dst}_{nrel}_{mode}",
    )


def kernel(x_Patient, x_Admission, edges, params):
    p = params
    sizes = {"Patient": x_Patient.shape[0], "Admission": x_Admission.shape[0]}
    for nt in ("Diagnosis", "Medication", "Procedure", "LabTest"):
        sizes[nt] = p["emb"][nt].shape[0]

    x = {
        "Patient": _encode(x_Patient, p["pat_lin"]["W"], p["pat_lin"]["b"],
                           p["pat_bn"]["g"], p["pat_bn"]["b"]),
        "Admission": _encode(x_Admission, p["adm_lin"]["W"], p["adm_lin"]["b"],
                             p["adm_bn"]["g"], p["adm_bn"]["b"]),
        "Diagnosis": p["emb"]["Diagnosis"],
        "Medication": p["emb"]["Medication"],
        "Procedure": p["emb"]["Procedure"],
        "LabTest": p["emb"]["LabTest"],
    }

    srcp, dstp, epad = {}, {}, {}
    for (s, d) in _REL_LIST:
        k = _rkey(s, d)
        e = edges[k].shape[1]
        e_pad = _round_up(e, EGROUP)
        pad = e_pad - e
        fill_src = (jnp.arange(pad, dtype=jnp.int32) % sizes[s])
        srcp[k] = jnp.concatenate([edges[k][0], fill_src])
        dstp[k] = jnp.concatenate(
            [edges[k][1], jnp.full((pad,), sizes[d], jnp.int32)])
        epad[k] = e_pad

    tok = jnp.zeros((16,), jnp.int32)
    parts, cnt = {}, {}
    for (s, d) in _REL_LIST:
        k = _rkey(s, d)
        parts[k], tok = _partition(srcp[k], dstp[k], sizes[d], tok)
        cnt[k], tok = _count(parts[k], sizes[d], epad[k], tok)

    rels_by_dst = {}
    for (s, d) in _REL_LIST:
        rels_by_dst.setdefault(d, []).append((s, d))

    out = {}
    for li, l in enumerate(("1", "2", "3")):
        seg = {}
        for (s, d) in _REL_LIST:
            k = _rkey(s, d)
            seg[k], tok = _segsum(parts[k], x[s], sizes[d], epad[k], tok)
        newx = {}
        for d, rels in rels_by_dst.items():
            n_d = sizes[d]
            args = [x[d]]
            for (s, _) in rels:
                k = _rkey(s, d)
                pc = p["conv"][l][k]
                args += [seg[k], cnt[k], pc["Wl"], pc["Wr"], pc["bl"].reshape(1, -1)]
            if li < 2:
                bn = p["bn"][l][d]
                args += [bn["g"].reshape(1, -1), bn["b"].reshape(1, -1)]
                newx[d] = _combine_kernel(n_d, len(rels), "bn")(*args)
            else:
                wh = jnp.pad(p["lin"][d]["W"], ((0, 0), (0, 128 - p["lin"][d]["W"].shape[1])))
                bh = jnp.pad(p["lin"][d]["b"], (0, 128 - p["lin"][d]["b"].shape[0]))
                args += [wh, bh.reshape(1, -1)]
                out[d] = _combine_kernel(n_d, len(rels), "head")(*args)
        x = newx

    nout = p["lin"]["Patient"]["W"].shape[1]
    return tuple(out[nt][:, :nout] for nt in _NTYPE_ORDER)
